# Initial kernel scaffold; baseline (speedup 1.0000x reference)
import jax
import jax.numpy as jnp
from jax.experimental import pallas as pl


def kernel(shop_id, item_id, category_1_id, brand_id, time_type, shop_id_list, item_id_list, time_type_list, rank_7, rank_30, rank_90, hours, price_list, hours_list, W_shop_id, W_item_id, W_category_1_id, W_brand_id, W_time_type, W_shop_id_list, W_item_id_list, W_time_type_list):
    raise NotImplementedError("write your pallas kernel here")


# speedup vs baseline: 229.6032x; 229.6032x over previous
import functools

import jax
import jax.numpy as jnp
from jax import lax
from jax.experimental import pallas as pl
from jax.experimental.pallas import tpu as pltpu
from jax.experimental.pallas import tpu_sc as plsc

B = 16384
L = 200
T = 100000
NC, NS = 2, 16
NW = NC * NS
ROWS_L = B // (NW // 2)
ROWS_R = B // NW
RC = 64
UNROLL = 8


def _sc_body(sll, ill, ttl, sid, iid, cid, bid, tid,
             w_sll, w_ill, w_ttl, w_shop, w_item, w_cat, w_brand, w_tt,
             out_shop, out_item, out_rest,
             tbl, idx_buf, lacc, racc, sidx, svals, ttab, tltab, sem):
    c = lax.axis_index("c")
    s = lax.axis_index("s")
    wid = s * NC + c
    is_a = wid < (NW // 2)
    not_a = jnp.logical_not(is_a)
    wl = jnp.where(is_a, wid, wid - NW // 2)
    lbase = wl * ROWS_L
    rbase = wid * ROWS_R

    @pl.when(is_a)
    def _():
        pltpu.sync_copy(w_sll, tbl)

    @pl.when(not_a)
    def _():
        pltpu.sync_copy(w_ill, tbl)

    pltpu.sync_copy(w_ttl, tltab)
    pltpu.sync_copy(w_tt, ttab)

    lane = lax.iota(jnp.int32, 16)
    lane_l = lane * L

    pltpu.sync_copy(tid.at[pl.ds(rbase, ROWS_R)], sidx)

    def tt_body(k, carry):
        iv = sidx[pl.ds(k * 16, 16)]
        v = plsc.load_gather(ttab, [iv])
        racc[pl.ds(k * 16, 16)] = jnp.where(iv != 0, v, 0.0)
        return carry

    lax.fori_loop(0, ROWS_R // 16, tt_body, 0)

    for idx_hbm, w_hbm in ((sid, w_shop), (iid, w_item), (cid, w_cat),
                           (bid, w_brand)):
        pltpu.sync_copy(idx_hbm.at[pl.ds(rbase, ROWS_R)], sidx)
        for j in range(ROWS_R // 128):
            pltpu.async_copy(w_hbm.at[sidx.at[pl.ds(j * 128, 128)]],
                             svals.at[pl.ds(j * 128, 128)], sem).wait()

        def s_body(k, carry):
            iv = sidx[pl.ds(k * 16, 16)]
            vv = svals[pl.ds(k * 16, 16)]
            racc[pl.ds(k * 16, 16)] = (racc[pl.ds(k * 16, 16)]
                                       + jnp.where(iv != 0, vv, 0.0))
            return carry

        lax.fori_loop(0, ROWS_R // 16, s_body, 0)

    def pooled(idx_flat, table_ref, n_rows, src_base, acc_ref, accumulate):
        def chunk_body(ci, carry):
            pltpu.sync_copy(idx_flat.at[pl.ds(src_base + ci * RC * L, RC * L)],
                            idx_buf)

            def grp_body(g, carry2):
                base = g * 16 * L

                def l_body(lb, acc):
                    for u in range(UNROLL):
                        off = base + lb * UNROLL + u
                        iv = plsc.load_gather(idx_buf, [lane_l + off])
                        gv = plsc.load_gather(table_ref, [iv])
                        acc = acc + jnp.where(iv != 0, gv, 0.0)
                    return acc

                acc = lax.fori_loop(0, L // UNROLL, l_body,
                                    jnp.zeros((16,), jnp.float32))
                o = ci * RC + g * 16
                if accumulate:
                    acc_ref[pl.ds(o, 16)] = acc_ref[pl.ds(o, 16)] + acc
                else:
                    acc_ref[pl.ds(o, 16)] = acc
                return carry2

            lax.fori_loop(0, RC // 16, grp_body, 0)
            return carry

        lax.fori_loop(0, n_rows // RC, chunk_body, 0)

    @pl.when(is_a)
    def _():
        pooled(sll, tbl, ROWS_L, lbase * L, lacc, False)

    @pl.when(not_a)
    def _():
        pooled(ill, tbl, ROWS_L, lbase * L, lacc, False)

    pooled(ttl, tltab, ROWS_R, rbase * L, racc, True)

    @pl.when(is_a)
    def _():
        pltpu.sync_copy(lacc, out_shop.at[pl.ds(lbase, ROWS_L)])

    @pl.when(not_a)
    def _():
        pltpu.sync_copy(lacc, out_item.at[pl.ds(lbase, ROWS_L)])

    pltpu.sync_copy(racc, out_rest.at[pl.ds(rbase, ROWS_R)])


_sc_call = pl.kernel(
    _sc_body,
    out_type=(jax.ShapeDtypeStruct((B,), jnp.float32),
              jax.ShapeDtypeStruct((B,), jnp.float32),
              jax.ShapeDtypeStruct((B,), jnp.float32)),
    mesh=plsc.VectorSubcoreMesh(core_axis_name="c", subcore_axis_name="s"),
    compiler_params=pltpu.CompilerParams(needs_layout_passes=False),
    scratch_types=[
        pltpu.VMEM((T,), jnp.float32),
        pltpu.VMEM((RC * L,), jnp.int32),
        pltpu.VMEM((ROWS_L,), jnp.float32),
        pltpu.VMEM((ROWS_R,), jnp.float32),
        pltpu.VMEM((ROWS_R,), jnp.int32),
        pltpu.VMEM((ROWS_R,), jnp.float32),
        pltpu.VMEM((128,), jnp.float32),
        pltpu.VMEM((128,), jnp.float32),
        pltpu.SemaphoreType.DMA,
    ],
)

BLK = 2048


def _tc_body(price_ref, hlist_ref, r7, r30, r90, hr, ps, pi_, pr, out_ref):
    srow = (jnp.sum(price_ref[...], axis=1, keepdims=True)
            + jnp.sum(hlist_ref[...], axis=1, keepdims=True))
    out_ref[...] = (srow + r7[...] + r30[...] + r90[...] + hr[...]
                    + ps[...] + pi_[...] + pr[...])


def _tc_call(price_list, hours_list, rank_7, rank_30, rank_90, hours,
             p_shop, p_item, p_rest):
    col = pl.BlockSpec((BLK, 1), lambda i: (i, 0))
    mat = pl.BlockSpec((BLK, L), lambda i: (i, 0))
    return pl.pallas_call(
        _tc_body,
        grid=(B // BLK,),
        in_specs=[mat, mat, col, col, col, col, col, col, col],
        out_specs=col,
        out_shape=jax.ShapeDtypeStruct((B, 1), jnp.float32),
    )(price_list, hours_list, rank_7, rank_30, rank_90, hours,
      p_shop, p_item, p_rest)


def kernel(shop_id, item_id, category_1_id, brand_id, time_type,
           shop_id_list, item_id_list, time_type_list,
           rank_7, rank_30, rank_90, hours, price_list, hours_list,
           W_shop_id, W_item_id, W_category_1_id, W_brand_id, W_time_type,
           W_shop_id_list, W_item_id_list, W_time_type_list):
    sll = shop_id_list.reshape(-1)
    ill = item_id_list.reshape(-1)
    ttl = time_type_list.reshape(-1)
    w_tt = jnp.pad(W_time_type.reshape(-1), (0, 128 - W_time_type.shape[0]))
    w_ttl = jnp.pad(W_time_type_list.reshape(-1),
                    (0, 128 - W_time_type_list.shape[0]))
    p_shop, p_item, p_rest = _sc_call(
        sll, ill, ttl,
        shop_id.astype(jnp.int32), item_id.astype(jnp.int32),
        category_1_id.astype(jnp.int32), brand_id.astype(jnp.int32),
        time_type.astype(jnp.int32),
        W_shop_id_list.reshape(-1), W_item_id_list.reshape(-1), w_ttl,
        W_shop_id.reshape(-1), W_item_id.reshape(-1),
        W_category_1_id.reshape(-1), W_brand_id.reshape(-1), w_tt)
    return _tc_call(price_list, hours_list, rank_7, rank_30, rank_90, hours,
                    p_shop.reshape(B, 1), p_item.reshape(B, 1),
                    p_rest.reshape(B, 1))
